# baseline (speedup 1.0000x reference)
import functools

import jax
import jax.numpy as jnp
from jax import lax
from jax.experimental import pallas as pl
from jax.experimental.pallas import tpu as pltpu
from jax.experimental.pallas import tpu_sc as plsc


def _edge_update_body(ea_ref, w_ref, b_ref, out_ref):
    ef = jnp.maximum(
        jnp.dot(ea_ref[...], w_ref[...], preferred_element_type=jnp.float32)
        + b_ref[...],
        0.0,
    )
    h = out_ref.shape[-1]
    out_ref[0] = ef[:, :h]
    out_ref[1] = ef[:, h:]


def _edge_update(edge_attr, W_edge, b_edge, out_rows=None, block_e=5000):
    E, D = edge_attr.shape
    H = W_edge.shape[1]
    Hh = H // 2
    grid = E // block_e
    out_rows = E if out_rows is None else out_rows
    return pl.pallas_call(
        _edge_update_body,
        grid=(grid,),
        in_specs=[
            pl.BlockSpec((block_e, D), lambda i: (i, 0)),
            pl.BlockSpec((D, H), lambda i: (0, 0)),
            pl.BlockSpec((1, H), lambda i: (0, 0)),
        ],
        out_specs=pl.BlockSpec((2, block_e, Hh), lambda i: (0, i, 0)),
        out_shape=jax.ShapeDtypeStruct((2, out_rows, Hh), jnp.float32),
    )(edge_attr, W_edge, b_edge.reshape(1, H))


def _make_segsum_sc(E_pad, N, N_acc, Hh, n_subcores=16, batch=128, nbuf=3):
    epw = E_pad // n_subcores
    n_iter = epw // batch
    assert n_iter * batch == epw and batch % 8 == 0 and batch <= 128
    row_chunk = 400
    n_chunks = N // row_chunk
    assert n_chunks * row_chunk == N and row_chunk % 8 == 0
    mesh = plsc.VectorSubcoreMesh(core_axis_name="c", subcore_axis_name="s")

    @functools.partial(
        pl.kernel,
        mesh=mesh,
        out_type=jax.ShapeDtypeStruct((2, N, Hh), jnp.float32),
        scratch_types=(
            [pltpu.VMEM((batch,), jnp.int32) for _ in range(nbuf)]
            + [pltpu.VMEM((batch, Hh), jnp.float32) for _ in range(nbuf)]
            + [pltpu.SemaphoreType.DMA for _ in range(2 * nbuf)]
            + [pltpu.VMEM_SHARED((N_acc, Hh), jnp.float32)]
        ),
    )
    def segsum(ef_hbm, row_hbm, zeros_hbm, out_hbm, *scratch):
        idxs = scratch[0:nbuf]
        efs = scratch[nbuf : 2 * nbuf]
        lsems = scratch[2 * nbuf : 3 * nbuf]
        ssems = scratch[3 * nbuf : 4 * nbuf]
        acc_sh = scratch[4 * nbuf]
        cid = lax.axis_index("c")
        sid = lax.axis_index("s")

        def zero_chunk(j):
            pltpu.sync_copy(
                zeros_hbm.at[pl.ds(j * row_chunk, row_chunk)],
                acc_sh.at[pl.ds(j * row_chunk, row_chunk)],
            )

        zero_chunk(sid)

        @pl.when(sid + n_subcores < n_chunks)
        def _():
            zero_chunk(sid + n_subcores)

        @pl.when(sid == 0)
        def _():
            pltpu.sync_copy(
                zeros_hbm.at[pl.ds(N, N_acc - N)], acc_sh.at[pl.ds(N, N_acc - N)]
            )

        plsc.subcore_barrier()

        def start_load(b, j):
            base = sid * epw + j * batch
            pltpu.async_copy(row_hbm.at[pl.ds(base, batch)], idxs[b], lsems[b])
            pltpu.async_copy(ef_hbm.at[cid, pl.ds(base, batch)], efs[b], lsems[b])

        def wait_load(b):
            pltpu.make_async_copy(row_hbm.at[pl.ds(0, batch)], idxs[b], lsems[b]).wait()
            pltpu.make_async_copy(ef_hbm.at[0, pl.ds(0, batch)], efs[b], lsems[b]).wait()

        def start_scatter(b):
            pltpu.async_copy(efs[b], acc_sh.at[idxs[b]], ssems[b], add=True)

        def wait_scatter(b):
            pltpu.make_async_copy(efs[b], acc_sh.at[idxs[b]], ssems[b]).wait()

        n_main = (n_iter // nbuf) * nbuf

        start_load(0, 0)
        start_load(1, 1)

        def group(g, carry):
            for b in range(nbuf):
                j = g * nbuf + b
                wait_load(b)
                start_scatter(b)
                b2 = (b + 2) % nbuf
                if b >= 1:
                    wait_scatter(b2)
                else:

                    @pl.when(g > 0)
                    def _():
                        wait_scatter(b2)

                @pl.when(j + 2 < n_iter)
                def _():
                    start_load(b2, j + 2)

            return carry

        lax.fori_loop(0, n_main // nbuf, group, 0)

        for j in range(n_main, n_iter):
            b = j % nbuf
            wait_load(b)
            start_scatter(b)
            wait_scatter((b + 2) % nbuf)
        wait_scatter((n_iter - 1) % nbuf)

        plsc.subcore_barrier()

        def write_chunk(j):
            pltpu.sync_copy(
                acc_sh.at[pl.ds(j * row_chunk, row_chunk)],
                out_hbm.at[cid, pl.ds(j * row_chunk, row_chunk)],
            )

        write_chunk(sid)

        @pl.when(sid + n_subcores < n_chunks)
        def _():
            write_chunk(sid + n_subcores)

    return segsum


def _node_update_body(x_ref, sa_ref, sb_ref, sc_ref, w_ref, b_ref, scale_ref, beta_ref, out_ref):
    s = (
        jnp.concatenate([sa_ref[0], sa_ref[1]], axis=1)
        + jnp.concatenate([sb_ref[0], sb_ref[1]], axis=1)
        + jnp.concatenate([sc_ref[0], sc_ref[1]], axis=1)
    )
    agg = x_ref[...] * s
    h = jnp.maximum(
        jnp.dot(agg, w_ref[...], preferred_element_type=jnp.float32) + b_ref[...],
        0.0,
    )
    out_ref[...] = h * scale_ref[...] + beta_ref[...]


def _node_update(x, s2a, s2b, s2c, W_node, b_node, scale, beta, block_n=2000):
    N, H = x.shape
    Hh = H // 2
    grid = N // block_n
    return pl.pallas_call(
        _node_update_body,
        grid=(grid,),
        in_specs=[
            pl.BlockSpec((block_n, H), lambda i: (i, 0)),
            pl.BlockSpec((2, block_n, Hh), lambda i: (0, i, 0)),
            pl.BlockSpec((2, block_n, Hh), lambda i: (0, i, 0)),
            pl.BlockSpec((2, block_n, Hh), lambda i: (0, i, 0)),
            pl.BlockSpec((H, H), lambda i: (0, 0)),
            pl.BlockSpec((1, H), lambda i: (0, 0)),
            pl.BlockSpec((1, H), lambda i: (0, 0)),
            pl.BlockSpec((1, H), lambda i: (0, 0)),
        ],
        out_specs=pl.BlockSpec((block_n, H), lambda i: (i, 0)),
        out_shape=jax.ShapeDtypeStruct((N, H), jnp.float32),
    )(x, s2a, s2b, s2c, W_node, b_node.reshape(1, H), scale.reshape(1, H), beta.reshape(1, H))


def kernel(x, edge_index, edge_attr, W_edge, b_edge, W_node, b_node, gamma, beta):
    N, H = x.shape
    E = edge_attr.shape[0]
    Hh = H // 2
    row = edge_index[0]

    E_chunk = 16 * 128 * 27
    N_acc = N + 8
    EA = E_chunk
    EC = E - 2 * E_chunk

    row_a = row[:EA]
    row_b = row[EA : 2 * EA]
    row_c = jnp.concatenate(
        [row[2 * EA :], jnp.full((E_chunk - EC,), N, dtype=row.dtype)]
    )

    ef_a = _edge_update(edge_attr[:EA], W_edge, b_edge, block_e=4608)
    ef_b = _edge_update(edge_attr[EA : 2 * EA], W_edge, b_edge, block_e=4608)
    ef_c = _edge_update(
        edge_attr[2 * EA :], W_edge, b_edge, out_rows=E_chunk, block_e=6176
    )

    zeros = jnp.zeros((N_acc, Hh), dtype=jnp.float32)
    segsum = _make_segsum_sc(E_chunk, N, N_acc, Hh)
    s2a = segsum(ef_a, row_a, zeros)
    s2b = segsum(ef_b, row_b, zeros)
    s2c = segsum(ef_c, row_c, zeros)

    scale = gamma * jax.lax.rsqrt(jnp.float32(1.0 + 1e-3))
    return _node_update(x, s2a, s2b, s2c, W_node, b_node, scale, beta)


# speedup vs baseline: 1.1038x; 1.1038x over previous
import functools

import jax
import jax.numpy as jnp
from jax import lax
from jax.experimental import pallas as pl
from jax.experimental.pallas import tpu as pltpu
from jax.experimental.pallas import tpu_sc as plsc


def _edge_update_body(ea_ref, w_ref, b_ref, out_ref):
    ef = jnp.maximum(
        jnp.dot(ea_ref[...], w_ref[...], preferred_element_type=jnp.float32)
        + b_ref[...],
        0.0,
    )
    h = out_ref.shape[-1]
    out_ref[0] = ef[:, :h]
    out_ref[1] = ef[:, h:]


def _edge_update(edge_attr, W_edge, b_edge, out_rows=None, block_e=5000):
    E, D = edge_attr.shape
    H = W_edge.shape[1]
    Hh = H // 2
    grid = E // block_e
    out_rows = E if out_rows is None else out_rows
    return pl.pallas_call(
        _edge_update_body,
        grid=(grid,),
        in_specs=[
            pl.BlockSpec((block_e, D), lambda i: (i, 0)),
            pl.BlockSpec((D, H), lambda i: (0, 0)),
            pl.BlockSpec((1, H), lambda i: (0, 0)),
        ],
        out_specs=pl.BlockSpec((2, block_e, Hh), lambda i: (0, i, 0)),
        out_shape=jax.ShapeDtypeStruct((2, out_rows, Hh), jnp.float32),
    )(edge_attr, W_edge, b_edge.reshape(1, H))


def _make_segsum_sc(E_pad, N, N_acc, Hh, n_subcores=16, batch=128, nbuf=3):
    epw = E_pad // n_subcores
    n_iter = epw // batch
    assert n_iter * batch == epw and batch % 8 == 0 and batch <= 128
    row_chunk = 400
    n_chunks = N // row_chunk
    assert n_chunks * row_chunk == N and row_chunk % 8 == 0
    mesh = plsc.VectorSubcoreMesh(core_axis_name="c", subcore_axis_name="s")

    @functools.partial(
        pl.kernel,
        mesh=mesh,
        out_type=jax.ShapeDtypeStruct((2, N, Hh), jnp.float32),
        scratch_types=(
            [pltpu.VMEM((batch,), jnp.int32) for _ in range(nbuf)]
            + [pltpu.VMEM((batch, Hh), jnp.float32) for _ in range(nbuf)]
            + [pltpu.SemaphoreType.DMA for _ in range(2 * nbuf)]
            + [pltpu.VMEM_SHARED((N_acc, Hh), jnp.float32)]
        ),
    )
    def segsum(ef_hbm, row_hbm, zeros_hbm, out_hbm, *scratch):
        idxs = scratch[0:nbuf]
        efs = scratch[nbuf : 2 * nbuf]
        lsems = scratch[2 * nbuf : 3 * nbuf]
        ssems = scratch[3 * nbuf : 4 * nbuf]
        acc_sh = scratch[4 * nbuf]
        cid = lax.axis_index("c")
        sid = lax.axis_index("s")

        def zero_chunk(j):
            pltpu.sync_copy(
                zeros_hbm.at[pl.ds(j * row_chunk, row_chunk)],
                acc_sh.at[pl.ds(j * row_chunk, row_chunk)],
            )

        zero_chunk(sid)

        @pl.when(sid + n_subcores < n_chunks)
        def _():
            zero_chunk(sid + n_subcores)

        @pl.when(sid == 0)
        def _():
            pltpu.sync_copy(
                zeros_hbm.at[pl.ds(N, N_acc - N)], acc_sh.at[pl.ds(N, N_acc - N)]
            )

        plsc.subcore_barrier()

        def start_load(b, j):
            base = sid * epw + j * batch
            pltpu.async_copy(row_hbm.at[pl.ds(base, batch)], idxs[b], lsems[b])
            pltpu.async_copy(ef_hbm.at[cid, pl.ds(base, batch)], efs[b], lsems[b])

        def wait_load(b):
            pltpu.make_async_copy(row_hbm.at[pl.ds(0, batch)], idxs[b], lsems[b]).wait()
            pltpu.make_async_copy(ef_hbm.at[0, pl.ds(0, batch)], efs[b], lsems[b]).wait()

        def start_scatter(b):
            pltpu.async_copy(efs[b], acc_sh.at[idxs[b]], ssems[b], add=True)

        def wait_scatter(b):
            pltpu.make_async_copy(efs[b], acc_sh.at[idxs[b]], ssems[b]).wait()

        n_main = (n_iter // nbuf) * nbuf

        start_load(0, 0)
        start_load(1, 1)

        def group(g, carry):
            for b in range(nbuf):
                j = g * nbuf + b
                wait_load(b)
                start_scatter(b)
                b2 = (b + 2) % nbuf
                if b >= 1:
                    wait_scatter(b2)
                else:

                    @pl.when(g > 0)
                    def _():
                        wait_scatter(b2)

                @pl.when(j + 2 < n_iter)
                def _():
                    start_load(b2, j + 2)

            return carry

        lax.fori_loop(0, n_main // nbuf, group, 0)

        for j in range(n_main, n_iter):
            b = j % nbuf
            wait_load(b)
            start_scatter(b)
            wait_scatter((b + 2) % nbuf)
        wait_scatter((n_iter - 1) % nbuf)

        plsc.subcore_barrier()

        def write_chunk(j):
            pltpu.sync_copy(
                acc_sh.at[pl.ds(j * row_chunk, row_chunk)],
                out_hbm.at[cid, pl.ds(j * row_chunk, row_chunk)],
            )

        write_chunk(sid)

        @pl.when(sid + n_subcores < n_chunks)
        def _():
            write_chunk(sid + n_subcores)

    return segsum


def _node_update_body(x_ref, sa_ref, sb_ref, w_ref, b_ref, scale_ref, beta_ref, out_ref):
    s = jnp.concatenate([sa_ref[0], sa_ref[1]], axis=1) + jnp.concatenate(
        [sb_ref[0], sb_ref[1]], axis=1
    )
    agg = x_ref[...] * s
    h = jnp.maximum(
        jnp.dot(agg, w_ref[...], preferred_element_type=jnp.float32) + b_ref[...],
        0.0,
    )
    out_ref[...] = h * scale_ref[...] + beta_ref[...]


def _node_update(x, s2a, s2b, W_node, b_node, scale, beta, block_n=2000):
    N, H = x.shape
    Hh = H // 2
    grid = N // block_n
    return pl.pallas_call(
        _node_update_body,
        grid=(grid,),
        in_specs=[
            pl.BlockSpec((block_n, H), lambda i: (i, 0)),
            pl.BlockSpec((2, block_n, Hh), lambda i: (0, i, 0)),
            pl.BlockSpec((2, block_n, Hh), lambda i: (0, i, 0)),
            pl.BlockSpec((H, H), lambda i: (0, 0)),
            pl.BlockSpec((1, H), lambda i: (0, 0)),
            pl.BlockSpec((1, H), lambda i: (0, 0)),
            pl.BlockSpec((1, H), lambda i: (0, 0)),
        ],
        out_specs=pl.BlockSpec((block_n, H), lambda i: (i, 0)),
        out_shape=jax.ShapeDtypeStruct((N, H), jnp.float32),
    )(x, s2a, s2b, W_node, b_node.reshape(1, H), scale.reshape(1, H), beta.reshape(1, H))


def kernel(x, edge_index, edge_attr, W_edge, b_edge, W_node, b_node, gamma, beta):
    N, H = x.shape
    E = edge_attr.shape[0]
    Hh = H // 2
    row = edge_index[0]

    E_chunk = 16 * 128 * 40
    N_acc = N + 8
    EA = E_chunk
    EB = E - EA

    row_a = row[:EA]
    row_b = jnp.concatenate(
        [row[EA:], jnp.full((E_chunk - EB,), N, dtype=row.dtype)]
    )

    ef_a = _edge_update(edge_attr[:EA], W_edge, b_edge, block_e=10240)
    ef_b = _edge_update(
        edge_attr[EA:], W_edge, b_edge, out_rows=E_chunk, block_e=9760
    )

    zeros = jnp.zeros((N_acc, Hh), dtype=jnp.float32)
    segsum = _make_segsum_sc(E_chunk, N, N_acc, Hh)
    s2a = segsum(ef_a, row_a, zeros)
    s2b = segsum(ef_b, row_b, zeros)

    scale = gamma * jax.lax.rsqrt(jnp.float32(1.0 + 1e-3))
    return _node_update(x, s2a, s2b, W_node, b_node, scale, beta)


# speedup vs baseline: 1.1070x; 1.0029x over previous
import functools

import jax
import jax.numpy as jnp
from jax import lax
from jax.experimental import pallas as pl
from jax.experimental.pallas import tpu as pltpu
from jax.experimental.pallas import tpu_sc as plsc


def _edge_update_body(ea_ref, w_ref, b_ref, out_ref):
    ef = jnp.maximum(
        jnp.dot(ea_ref[...], w_ref[...], preferred_element_type=jnp.float32)
        + b_ref[...],
        0.0,
    )
    h = out_ref.shape[-1]
    out_ref[0] = ef[:, :h]
    out_ref[1] = ef[:, h:]


def _edge_update(edge_attr, W_edge, b_edge, out_rows=None, block_e=5000):
    E, D = edge_attr.shape
    H = W_edge.shape[1]
    Hh = H // 2
    grid = E // block_e
    out_rows = E if out_rows is None else out_rows
    return pl.pallas_call(
        _edge_update_body,
        grid=(grid,),
        in_specs=[
            pl.BlockSpec((block_e, D), lambda i: (i, 0)),
            pl.BlockSpec((D, H), lambda i: (0, 0)),
            pl.BlockSpec((1, H), lambda i: (0, 0)),
        ],
        out_specs=pl.BlockSpec((2, block_e, Hh), lambda i: (0, i, 0)),
        out_shape=jax.ShapeDtypeStruct((2, out_rows, Hh), jnp.float32),
    )(edge_attr, W_edge, b_edge.reshape(1, H))


def _make_segsum_sc(E_pad, N, N_acc, Hh, n_subcores=16, batch=128, nbuf=3):
    epw = E_pad // n_subcores
    n_iter = epw // batch
    assert n_iter * batch == epw and batch % 8 == 0 and batch <= 128
    row_chunk = 400
    n_chunks = N // row_chunk
    assert n_chunks * row_chunk == N and row_chunk % 8 == 0
    mesh = plsc.VectorSubcoreMesh(core_axis_name="c", subcore_axis_name="s")

    @functools.partial(
        pl.kernel,
        mesh=mesh,
        out_type=jax.ShapeDtypeStruct((2, N, Hh), jnp.float32),
        scratch_types=(
            [pltpu.VMEM((batch,), jnp.int32) for _ in range(nbuf)]
            + [pltpu.VMEM((batch, Hh), jnp.float32) for _ in range(nbuf)]
            + [pltpu.SemaphoreType.DMA for _ in range(2 * nbuf)]
            + [pltpu.VMEM_SHARED((N_acc, Hh), jnp.float32)]
        ),
    )
    def segsum(ef_hbm, row_hbm, zeros_hbm, out_hbm, *scratch):
        idxs = scratch[0:nbuf]
        efs = scratch[nbuf : 2 * nbuf]
        lsems = scratch[2 * nbuf : 3 * nbuf]
        ssems = scratch[3 * nbuf : 4 * nbuf]
        acc_sh = scratch[4 * nbuf]
        cid = lax.axis_index("c")
        sid = lax.axis_index("s")

        def zero_chunk(j):
            pltpu.sync_copy(
                zeros_hbm.at[pl.ds(j * row_chunk, row_chunk)],
                acc_sh.at[pl.ds(j * row_chunk, row_chunk)],
            )

        zero_chunk(sid)

        @pl.when(sid + n_subcores < n_chunks)
        def _():
            zero_chunk(sid + n_subcores)

        @pl.when(sid == 0)
        def _():
            pltpu.sync_copy(
                zeros_hbm.at[pl.ds(N, N_acc - N)], acc_sh.at[pl.ds(N, N_acc - N)]
            )

        plsc.subcore_barrier()

        def start_load(b, j):
            base = sid * epw + j * batch
            pltpu.async_copy(row_hbm.at[pl.ds(base, batch)], idxs[b], lsems[b])
            pltpu.async_copy(ef_hbm.at[cid, pl.ds(base, batch)], efs[b], lsems[b])

        def wait_load(b):
            pltpu.make_async_copy(row_hbm.at[pl.ds(0, batch)], idxs[b], lsems[b]).wait()
            pltpu.make_async_copy(ef_hbm.at[0, pl.ds(0, batch)], efs[b], lsems[b]).wait()

        def start_scatter(b):
            pltpu.async_copy(efs[b], acc_sh.at[idxs[b]], ssems[b], add=True)

        def wait_scatter(b):
            pltpu.make_async_copy(efs[b], acc_sh.at[idxs[b]], ssems[b]).wait()

        n_main = (n_iter // nbuf) * nbuf

        start_load(0, 0)
        start_load(1, 1)

        def group(g, carry):
            for b in range(nbuf):
                j = g * nbuf + b
                wait_load(b)
                start_scatter(b)
                b2 = (b + 2) % nbuf
                if b >= 1:
                    wait_scatter(b2)
                else:

                    @pl.when(g > 0)
                    def _():
                        wait_scatter(b2)

                @pl.when(j + 2 < n_iter)
                def _():
                    start_load(b2, j + 2)

            return carry

        lax.fori_loop(0, n_main // nbuf, group, 0)

        for j in range(n_main, n_iter):
            b = j % nbuf
            wait_load(b)
            start_scatter(b)
            wait_scatter((b + 2) % nbuf)
        wait_scatter((n_iter - 1) % nbuf)

        plsc.subcore_barrier()

        def write_chunk(j):
            pltpu.sync_copy(
                acc_sh.at[pl.ds(j * row_chunk, row_chunk)],
                out_hbm.at[cid, pl.ds(j * row_chunk, row_chunk)],
            )

        write_chunk(sid)

        @pl.when(sid + n_subcores < n_chunks)
        def _():
            write_chunk(sid + n_subcores)

    return segsum


def _node_update_body(x_ref, sa_ref, sb_ref, w_ref, b_ref, scale_ref, beta_ref, out_ref):
    s = jnp.concatenate([sa_ref[0], sa_ref[1]], axis=1) + jnp.concatenate(
        [sb_ref[0], sb_ref[1]], axis=1
    )
    agg = x_ref[...] * s
    h = jnp.maximum(
        jnp.dot(agg, w_ref[...], preferred_element_type=jnp.float32) + b_ref[...],
        0.0,
    )
    out_ref[...] = h * scale_ref[...] + beta_ref[...]


def _node_update(x, s2a, s2b, W_node, b_node, scale, beta, block_n=2000):
    N, H = x.shape
    Hh = H // 2
    grid = N // block_n
    return pl.pallas_call(
        _node_update_body,
        grid=(grid,),
        in_specs=[
            pl.BlockSpec((block_n, H), lambda i: (i, 0)),
            pl.BlockSpec((2, block_n, Hh), lambda i: (0, i, 0)),
            pl.BlockSpec((2, block_n, Hh), lambda i: (0, i, 0)),
            pl.BlockSpec((H, H), lambda i: (0, 0)),
            pl.BlockSpec((1, H), lambda i: (0, 0)),
            pl.BlockSpec((1, H), lambda i: (0, 0)),
            pl.BlockSpec((1, H), lambda i: (0, 0)),
        ],
        out_specs=pl.BlockSpec((block_n, H), lambda i: (i, 0)),
        out_shape=jax.ShapeDtypeStruct((N, H), jnp.float32),
    )(x, s2a, s2b, W_node, b_node.reshape(1, H), scale.reshape(1, H), beta.reshape(1, H))


def kernel(x, edge_index, edge_attr, W_edge, b_edge, W_node, b_node, gamma, beta):
    N, H = x.shape
    E = edge_attr.shape[0]
    Hh = H // 2
    row = edge_index[0]

    E_chunk = 16 * 128 * 40
    N_acc = N + 8
    EA = E_chunk
    EB = E - EA

    row_a = row[:EA]
    row_b = jnp.concatenate(
        [row[EA:], jnp.full((E_chunk - EB,), N, dtype=row.dtype)]
    )

    ef_a = _edge_update(edge_attr[:EA], W_edge, b_edge, block_e=16384)
    ef_b = _edge_update(
        edge_attr[EA:], W_edge, b_edge, out_rows=E_chunk, block_e=15616
    )

    zeros = jnp.zeros((N_acc, Hh), dtype=jnp.float32)
    segsum = _make_segsum_sc(E_chunk, N, N_acc, Hh)
    s2a = segsum(ef_a, row_a, zeros)
    s2b = segsum(ef_b, row_b, zeros)

    scale = gamma * jax.lax.rsqrt(jnp.float32(1.0 + 1e-3))
    return _node_update(x, s2a, s2b, W_node, b_node, scale, beta)
